# Initial kernel scaffold; baseline (speedup 1.0000x reference)
import jax
import jax.numpy as jnp
from jax.experimental import pallas as pl


def kernel(x_claim, x_entity, edge_index_cm, edge_index_ec, claim_batch_idx, W1_cm_l, b1_cm, W1_cm_r, W1_ec_l, b1_ec, W1_ec_r, W2_cm_l, b2_cm, W2_cm_r, W2_ec_l, b2_ec, W2_ec_r, Wc, bc):
    raise NotImplementedError("write your pallas kernel here")


# speedup vs baseline: 3.3905x; 3.3905x over previous
import functools

import jax
import jax.numpy as jnp
from jax import lax
from jax.experimental import pallas as pl
from jax.experimental.pallas import tpu as pltpu
from jax.experimental.pallas import tpu_sc as plsc

N_NODE = 50000
N_EDGE = 400000
DIM_IN = 128
DIM_H = 64
BATCH = 1024
CHUNK = 128
NTILE = 16
NSC = 2
EPAD = 401408
TRASH = N_NODE
NACC = 50176
HROWS = 3200
RBLK = 400


def _seg_sum_feat_split(table, srcp, dstp):
    n_per_tile = EPAD // NTILE
    n_chunks = n_per_tile // CHUNK
    rows_per_tile = NACC // NTILE

    def body(table_ref, src_ref, dst_ref, zeros_ref, out_ref,
             acc_sh, idx_v, dst_v, stage_v, sem):
        cid = lax.axis_index("c")
        sid = lax.axis_index("s")
        shift = cid * N_NODE

        tile_r0 = sid * rows_per_tile
        pltpu.sync_copy(zeros_ref.at[pl.ds(tile_r0, rows_per_tile)],
                        acc_sh.at[pl.ds(tile_r0, rows_per_tile)])
        plsc.subcore_barrier()

        e0 = sid * n_per_tile

        def step(i, _):
            off = e0 + i * CHUNK
            pltpu.sync_copy(src_ref.at[pl.ds(off, CHUNK)], idx_v)
            pltpu.sync_copy(dst_ref.at[pl.ds(off, CHUNK)], dst_v)
            for k in range(CHUNK // 16):
                sl = pl.ds(k * 16, 16)
                idx_v[sl] = idx_v[sl] + shift
            pltpu.async_copy(table_ref.at[idx_v], stage_v, sem).wait()
            pltpu.sync_copy(stage_v, acc_sh.at[dst_v], add=True)
            return 0

        lax.fori_loop(0, n_chunks, step, 0)
        plsc.subcore_barrier()

        @pl.when(sid < NTILE - 1)
        def _():
            r0 = sid * 3128
            pltpu.sync_copy(acc_sh.at[pl.ds(r0, 3128)],
                            out_ref.at[pl.ds(cid * N_NODE + r0, 3128)])

        @pl.when(sid == NTILE - 1)
        def _():
            r0 = (NTILE - 1) * 3128
            pltpu.sync_copy(acc_sh.at[pl.ds(r0, N_NODE - r0)],
                            out_ref.at[pl.ds(cid * N_NODE + r0, N_NODE - r0)])

    call = pl.kernel(
        body,
        out_type=jax.ShapeDtypeStruct((2 * N_NODE, 32), jnp.float32),
        mesh=plsc.VectorSubcoreMesh(core_axis_name="c", subcore_axis_name="s"),
        compiler_params=pltpu.CompilerParams(needs_layout_passes=False, use_tc_tiling_on_sc=False),
        scratch_types=[
            pltpu.VMEM_SHARED((NACC, 32), jnp.float32),
            pltpu.VMEM((CHUNK,), jnp.int32),
            pltpu.VMEM((CHUNK,), jnp.int32),
            pltpu.VMEM((CHUNK, 32), jnp.float32),
            pltpu.SemaphoreType.DMA,
        ],
    )
    return call(table, srcp, dstp, jnp.zeros((NACC, 32), jnp.float32))


def _degree_counts(dst_cm, dst_ec):
    nw = NSC * NTILE
    n_per_worker = EPAD // nw
    c2 = 448
    n_chunks = n_per_worker // c2
    hn = HROWS * 16

    def body(dcm_ref, dec_ref, ocm_ref, oec_ref, d_v, hcm_v, hec_v):
        wid = lax.axis_index("c") * NTILE + lax.axis_index("s")

        def zero(i, _):
            hcm_v[pl.ds(i * 16, 16)] = jnp.zeros((16,), jnp.float32)
            hec_v[pl.ds(i * 16, 16)] = jnp.zeros((16,), jnp.float32)
        lax.fori_loop(0, hn // 16, zero, None)

        e0 = wid * n_per_worker
        ones = jnp.ones((16,), jnp.float32)

        def step(i, _):
            off = e0 + i * c2
            pltpu.sync_copy(dcm_ref.at[pl.ds(off, c2)], d_v)
            for k in range(c2 // 16):
                plsc.addupdate_scatter(hcm_v, [d_v[pl.ds(k * 16, 16)]], ones)
            pltpu.sync_copy(dec_ref.at[pl.ds(off, c2)], d_v)
            for k in range(c2 // 16):
                plsc.addupdate_scatter(hec_v, [d_v[pl.ds(k * 16, 16)]], ones)
            return 0

        lax.fori_loop(0, n_chunks, step, 0)
        pltpu.sync_copy(hcm_v, ocm_ref.at[pl.ds(wid * hn, hn)])
        pltpu.sync_copy(hec_v, oec_ref.at[pl.ds(wid * hn, hn)])

    call = pl.kernel(
        body,
        out_type=(jax.ShapeDtypeStruct((nw * hn,), jnp.float32),
                  jax.ShapeDtypeStruct((nw * hn,), jnp.float32)),
        mesh=plsc.VectorSubcoreMesh(core_axis_name="c", subcore_axis_name="s"),
        compiler_params=pltpu.CompilerParams(needs_layout_passes=False, use_tc_tiling_on_sc=False),
        scratch_types=[
            pltpu.VMEM((448,), jnp.int32),
            pltpu.VMEM((HROWS * 16,), jnp.float32),
            pltpu.VMEM((HROWS * 16,), jnp.float32),
        ],
    )
    return call(dst_cm, dst_ec)


def _seg_sum_narrow(table, srcp, dstp):
    n_per_worker = EPAD // (NSC * NTILE)
    n_chunks = n_per_worker // CHUNK
    rows_per_tile = NACC // NTILE

    def body(table_ref, src_ref, dst_ref, zeros_ref, out_ref,
             acc_sh, idx_v, dst_v, stage_v, sem):
        cid = lax.axis_index("c")
        sid = lax.axis_index("s")

        tile_r0 = sid * rows_per_tile
        pltpu.sync_copy(zeros_ref.at[pl.ds(tile_r0, rows_per_tile)],
                        acc_sh.at[pl.ds(tile_r0, rows_per_tile)])
        plsc.subcore_barrier()

        e0 = (cid * NTILE + sid) * n_per_worker

        def step(i, _):
            off = e0 + i * CHUNK
            pltpu.sync_copy(src_ref.at[pl.ds(off, CHUNK)], idx_v)
            pltpu.sync_copy(dst_ref.at[pl.ds(off, CHUNK)], dst_v)
            pltpu.async_copy(table_ref.at[idx_v], stage_v, sem).wait()
            pltpu.sync_copy(stage_v, acc_sh.at[dst_v], add=True)
            return 0

        lax.fori_loop(0, n_chunks, step, 0)
        plsc.subcore_barrier()

        @pl.when(sid < NTILE - 1)
        def _():
            r0 = sid * 3128
            pltpu.sync_copy(acc_sh.at[pl.ds(r0, 3128)],
                            out_ref.at[pl.ds(cid * N_NODE + r0, 3128)])

        @pl.when(sid == NTILE - 1)
        def _():
            r0 = (NTILE - 1) * 3128
            pltpu.sync_copy(acc_sh.at[pl.ds(r0, N_NODE - r0)],
                            out_ref.at[pl.ds(cid * N_NODE + r0, N_NODE - r0)])

    call = pl.kernel(
        body,
        out_type=jax.ShapeDtypeStruct((2 * N_NODE, 16), jnp.float32),
        mesh=plsc.VectorSubcoreMesh(core_axis_name="c", subcore_axis_name="s"),
        compiler_params=pltpu.CompilerParams(needs_layout_passes=False, use_tc_tiling_on_sc=False),
        scratch_types=[
            pltpu.VMEM_SHARED((NACC, 16), jnp.float32),
            pltpu.VMEM((CHUNK,), jnp.int32),
            pltpu.VMEM((CHUNK,), jnp.int32),
            pltpu.VMEM((CHUNK, 16), jnp.float32),
            pltpu.SemaphoreType.DMA,
        ],
    )
    return call(table, srcp, dstp, jnp.zeros((NACC, 16), jnp.float32))


def _gather_rows(table, idx):
    per_w = BATCH // (NSC * NTILE)

    def body(table_ref, idx_ref, out_ref, idx_v, rows_v, sem):
        w = lax.axis_index("c") * NTILE + lax.axis_index("s")
        base = w * per_w
        pltpu.sync_copy(idx_ref.at[pl.ds(base, per_w)], idx_v)
        pltpu.async_copy(table_ref.at[idx_v], rows_v, sem).wait()
        pltpu.sync_copy(rows_v, out_ref.at[pl.ds(base, per_w)])

    call = pl.kernel(
        body,
        out_type=jax.ShapeDtypeStruct((BATCH, 16), jnp.float32),
        mesh=plsc.VectorSubcoreMesh(core_axis_name="c", subcore_axis_name="s"),
        compiler_params=pltpu.CompilerParams(needs_layout_passes=False, use_tc_tiling_on_sc=False),
        scratch_types=[
            pltpu.VMEM((per_w,), jnp.int32),
            pltpu.VMEM((per_w, 16), jnp.float32),
            pltpu.SemaphoreType.DMA,
        ],
    )
    return call(table, idx)


def _proj_stacked(x, w):
    n, d = x.shape
    wstk = jnp.stack([w[:, :32], w[:, 32:]])

    def body(x_ref, w_ref, o_ref):
        o_ref[...] = jnp.dot(x_ref[...], w_ref[0],
                             preferred_element_type=jnp.float32)

    nb = n // RBLK
    return pl.pallas_call(
        body,
        grid=(2, nb),
        in_specs=[pl.BlockSpec((RBLK, d), lambda c, i: (i, 0)),
                  pl.BlockSpec((1, d, 32), lambda c, i: (c, 0, 0))],
        out_specs=pl.BlockSpec((RBLK, 32), lambda c, i: (c * nb + i, 0)),
        out_shape=jax.ShapeDtypeStruct((2 * n, 32), jnp.float32),
    )(x, wstk)


def _hist_reduce(hist):
    t, m = hist.shape

    def body(h_ref, o_ref):
        o_ref[...] = jnp.sum(h_ref[...], axis=0, keepdims=True)

    return pl.pallas_call(
        body,
        grid=(m // 6400,),
        in_specs=[pl.BlockSpec((t, 6400), lambda i: (0, i))],
        out_specs=pl.BlockSpec((1, 6400), lambda i: (0, i)),
        out_shape=jax.ShapeDtypeStruct((1, m), jnp.float32),
    )(hist)


def _combine_relu(sa, sb, cnt, x, wr, b):
    n, d = x.shape

    def body(sa_ref, sb_ref, c_ref, x_ref, w_ref, b_ref, o_ref):
        c = jnp.clip(c_ref[...], 1.0)
        m = jnp.concatenate([sa_ref[...], sb_ref[...]], axis=1) / c
        o_ref[...] = jnp.maximum(
            m + b_ref[...] + jnp.dot(x_ref[...], w_ref[...],
                                     preferred_element_type=jnp.float32), 0.0)

    return pl.pallas_call(
        body,
        grid=(n // RBLK,),
        in_specs=[pl.BlockSpec((RBLK, 32), lambda i: (i, 0)),
                  pl.BlockSpec((RBLK, 32), lambda i: (i, 0)),
                  pl.BlockSpec((RBLK, 1), lambda i: (i, 0)),
                  pl.BlockSpec((RBLK, d), lambda i: (i, 0)),
                  pl.BlockSpec((d, DIM_H), lambda i: (0, 0)),
                  pl.BlockSpec((1, DIM_H), lambda i: (0, 0))],
        out_specs=pl.BlockSpec((RBLK, DIM_H), lambda i: (i, 0)),
        out_shape=jax.ShapeDtypeStruct((n, DIM_H), jnp.float32),
    )(sa, sb, cnt, x, wr, b)


def _proj_fold(x, w1, w2):
    n = x.shape[0]

    def body(x_ref, w1_ref, w2_ref, o_ref):
        w = jnp.dot(w1_ref[...], w2_ref[...], preferred_element_type=jnp.float32)
        o_ref[...] = jnp.dot(x_ref[...], w, preferred_element_type=jnp.float32)

    return pl.pallas_call(
        body,
        grid=(n // RBLK,),
        in_specs=[pl.BlockSpec((RBLK, DIM_H), lambda i: (i, 0)),
                  pl.BlockSpec((DIM_H, DIM_H), lambda i: (0, 0)),
                  pl.BlockSpec((DIM_H, 16), lambda i: (0, 0))],
        out_specs=pl.BlockSpec((RBLK, 16), lambda i: (i, 0)),
        out_shape=jax.ShapeDtypeStruct((n, 16), jnp.float32),
    )(x, w1, w2)


def _final_dense(s3a, s3b, hist, hc, wr2, wcp, b2, bcp):
    n = hc.shape[0]

    def body(sa_ref, sb_ref, c_ref, hc_ref, wr_ref, wc_ref, b2_ref, bc_ref,
             o_ref):
        c = jnp.clip(c_ref[...], 1.0)
        m = (sa_ref[...] + sb_ref[...]) / c
        fold = jnp.dot(wr_ref[...], wc_ref[...],
                       preferred_element_type=jnp.float32)
        bias = jnp.dot(b2_ref[...], wc_ref[...],
                       preferred_element_type=jnp.float32) + bc_ref[...]
        o_ref[...] = m + bias + jnp.dot(hc_ref[...], fold,
                                        preferred_element_type=jnp.float32)

    return pl.pallas_call(
        body,
        grid=(n // RBLK,),
        in_specs=[pl.BlockSpec((RBLK, 16), lambda i: (i, 0)),
                  pl.BlockSpec((RBLK, 16), lambda i: (i, 0)),
                  pl.BlockSpec((RBLK, 1), lambda i: (i, 0)),
                  pl.BlockSpec((RBLK, DIM_H), lambda i: (i, 0)),
                  pl.BlockSpec((DIM_H, DIM_H), lambda i: (0, 0)),
                  pl.BlockSpec((DIM_H, 16), lambda i: (0, 0)),
                  pl.BlockSpec((1, DIM_H), lambda i: (0, 0)),
                  pl.BlockSpec((1, 16), lambda i: (0, 0))],
        out_specs=pl.BlockSpec((RBLK, 16), lambda i: (i, 0)),
        out_shape=jax.ShapeDtypeStruct((n, 16), jnp.float32),
    )(s3a, s3b, hist, hc, wr2, wcp, b2, bcp)


def kernel(x_claim, x_entity, edge_index_cm, edge_index_ec, claim_batch_idx,
           W1_cm_l, b1_cm, W1_cm_r, W1_ec_l, b1_ec, W1_ec_r,
           W2_cm_l, b2_cm, W2_cm_r, W2_ec_l, b2_ec, W2_ec_r,
           Wc, bc):
    pad = EPAD - N_EDGE
    i32 = jnp.int32

    def pad_edges(ei):
        src = jnp.concatenate([ei[0].astype(i32), jnp.zeros((pad,), i32)])
        dst = jnp.concatenate([ei[1].astype(i32), jnp.full((pad,), TRASH, i32)])
        return src, dst

    src_cm, dst_cm = pad_edges(edge_index_cm)
    src_ec, dst_ec = pad_edges(edge_index_ec)

    wcp = jnp.pad(Wc, ((0, 0), (0, 14)))
    bcp = jnp.pad(bc, (0, 14)).reshape(1, 16)

    p1c = _proj_stacked(x_claim, W1_cm_l)
    p1e = _proj_stacked(x_entity, W1_ec_l)
    s1 = _seg_sum_feat_split(p1c, src_cm, dst_cm)
    s2 = _seg_sum_feat_split(p1e, src_ec, dst_ec)
    hist_cm, hist_ec = _degree_counts(dst_cm, dst_ec)
    cnt_cm = _hist_reduce(hist_cm.reshape(NSC * NTILE, -1))
    cnt_ec = _hist_reduce(hist_ec.reshape(NSC * NTILE, -1))
    cnt_cm = cnt_cm.reshape(-1)[:N_NODE].reshape(N_NODE, 1)
    cnt_ec = cnt_ec.reshape(-1)[:N_NODE].reshape(N_NODE, 1)
    h_e = _combine_relu(s1[:N_NODE], s1[N_NODE:], cnt_cm, x_entity, W1_cm_r,
                        b1_cm.reshape(1, -1))
    h_c = _combine_relu(s2[:N_NODE], s2[N_NODE:], cnt_ec, x_claim, W1_ec_r,
                        b1_ec.reshape(1, -1))

    p2 = _proj_fold(h_e, W2_ec_l, wcp)
    s3 = _seg_sum_narrow(p2, src_ec, dst_ec)
    out16 = _final_dense(s3[:N_NODE], s3[N_NODE:], cnt_ec, h_c, W2_ec_r, wcp,
                         b2_ec.reshape(1, -1), bcp)
    res = _gather_rows(out16, claim_batch_idx.astype(i32))
    return res[:, :2]


# speedup vs baseline: 4.5276x; 1.3354x over previous
import functools

import jax
import jax.numpy as jnp
from jax import lax
from jax.experimental import pallas as pl
from jax.experimental.pallas import tpu as pltpu
from jax.experimental.pallas import tpu_sc as plsc

N_NODE = 50000
N_EDGE = 400000
DIM_IN = 128
DIM_H = 64
BATCH = 1024
CHUNK = 128
NTILE = 16
NSC = 2
EPAD = 401408
TRASH = N_NODE
NACC = 50176
HROWS = 3200
RBLK = 400


def _seg_sum_feat_split(table, srcp, dstp):
    SUP = 4
    sup = SUP * CHUNK
    n_per_tile = EPAD // NTILE
    n_chunks = n_per_tile // sup
    rows_per_tile = NACC // NTILE

    def body(table_ref, src_ref, dst_ref, zeros_ref, out_ref,
             acc_sh, idx_v, dst_v, stage_v, gsem, ssem):
        cid = lax.axis_index("c")
        sid = lax.axis_index("s")
        shift = cid * N_NODE

        tile_r0 = sid * rows_per_tile
        pltpu.sync_copy(zeros_ref.at[pl.ds(tile_r0, rows_per_tile)],
                        acc_sh.at[pl.ds(tile_r0, rows_per_tile)])
        plsc.subcore_barrier()

        e0 = sid * n_per_tile

        def step(i, _):
            off = e0 + i * sup
            pltpu.sync_copy(src_ref.at[pl.ds(off, sup)], idx_v)
            pltpu.sync_copy(dst_ref.at[pl.ds(e0 // CHUNK + i * SUP, SUP)],
                            dst_v)
            for k in range(sup // 16):
                sl = pl.ds(k * 16, 16)
                idx_v[sl] = idx_v[sl] + shift
            gs = [pltpu.async_copy(
                      table_ref.at[idx_v.at[pl.ds(j * CHUNK, CHUNK)]],
                      stage_v.at[j], gsem) for j in range(SUP)]
            for g in gs:
                g.wait()
            ss = [pltpu.async_copy(stage_v.at[j], acc_sh.at[dst_v.at[j]],
                                   ssem, add=True) for j in range(SUP)]
            for sc in ss:
                sc.wait()
            return 0

        lax.fori_loop(0, n_chunks, step, 0)
        plsc.subcore_barrier()

        @pl.when(sid < NTILE - 1)
        def _():
            r0 = sid * 3128
            pltpu.sync_copy(acc_sh.at[pl.ds(r0, 3128)],
                            out_ref.at[pl.ds(cid * N_NODE + r0, 3128)])

        @pl.when(sid == NTILE - 1)
        def _():
            r0 = (NTILE - 1) * 3128
            pltpu.sync_copy(acc_sh.at[pl.ds(r0, N_NODE - r0)],
                            out_ref.at[pl.ds(cid * N_NODE + r0, N_NODE - r0)])

    call = pl.kernel(
        body,
        out_type=jax.ShapeDtypeStruct((2 * N_NODE, 32), jnp.float32),
        mesh=plsc.VectorSubcoreMesh(core_axis_name="c", subcore_axis_name="s"),
        compiler_params=pltpu.CompilerParams(needs_layout_passes=False, use_tc_tiling_on_sc=False),
        scratch_types=[
            pltpu.VMEM_SHARED((NACC, 32), jnp.float32),
            pltpu.VMEM((4 * CHUNK,), jnp.int32),
            pltpu.VMEM((4, CHUNK), jnp.int32),
            pltpu.VMEM((4, CHUNK, 32), jnp.float32),
            pltpu.SemaphoreType.DMA,
            pltpu.SemaphoreType.DMA,
        ],
    )
    return call(table, srcp, dstp.reshape(EPAD // CHUNK, CHUNK),
                jnp.zeros((NACC, 32), jnp.float32))


def _degree_counts(dst_cm, dst_ec):
    nw = NSC * NTILE
    n_per_worker = EPAD // nw
    c2 = 448
    n_chunks = n_per_worker // c2
    hn = HROWS * 16

    def body(dcm_ref, dec_ref, ocm_ref, oec_ref, d_v, hcm_v, hec_v):
        wid = lax.axis_index("c") * NTILE + lax.axis_index("s")

        def zero(i, _):
            hcm_v[pl.ds(i * 16, 16)] = jnp.zeros((16,), jnp.float32)
            hec_v[pl.ds(i * 16, 16)] = jnp.zeros((16,), jnp.float32)
        lax.fori_loop(0, hn // 16, zero, None)

        e0 = wid * n_per_worker
        ones = jnp.ones((16,), jnp.float32)

        def step(i, _):
            off = e0 + i * c2
            pltpu.sync_copy(dcm_ref.at[pl.ds(off, c2)], d_v)
            for k in range(c2 // 16):
                plsc.addupdate_scatter(hcm_v, [d_v[pl.ds(k * 16, 16)]], ones)
            pltpu.sync_copy(dec_ref.at[pl.ds(off, c2)], d_v)
            for k in range(c2 // 16):
                plsc.addupdate_scatter(hec_v, [d_v[pl.ds(k * 16, 16)]], ones)
            return 0

        lax.fori_loop(0, n_chunks, step, 0)
        pltpu.sync_copy(hcm_v, ocm_ref.at[pl.ds(wid * hn, hn)])
        pltpu.sync_copy(hec_v, oec_ref.at[pl.ds(wid * hn, hn)])

    call = pl.kernel(
        body,
        out_type=(jax.ShapeDtypeStruct((nw * hn,), jnp.float32),
                  jax.ShapeDtypeStruct((nw * hn,), jnp.float32)),
        mesh=plsc.VectorSubcoreMesh(core_axis_name="c", subcore_axis_name="s"),
        compiler_params=pltpu.CompilerParams(needs_layout_passes=False, use_tc_tiling_on_sc=False),
        scratch_types=[
            pltpu.VMEM((448,), jnp.int32),
            pltpu.VMEM((HROWS * 16,), jnp.float32),
            pltpu.VMEM((HROWS * 16,), jnp.float32),
        ],
    )
    return call(dst_cm, dst_ec)


def _seg_sum_narrow(table, srcp, dstp):
    SUP = 7
    sup = SUP * CHUNK
    n_per_worker = EPAD // (NSC * NTILE)
    n_chunks = n_per_worker // sup
    rows_per_tile = NACC // NTILE

    def body(table_ref, src_ref, dst_ref, zeros_ref, out_ref,
             acc_sh, idx_v, dst_v, stage_v, gsem, ssem):
        cid = lax.axis_index("c")
        sid = lax.axis_index("s")

        tile_r0 = sid * rows_per_tile
        pltpu.sync_copy(zeros_ref.at[pl.ds(tile_r0, rows_per_tile)],
                        acc_sh.at[pl.ds(tile_r0, rows_per_tile)])
        plsc.subcore_barrier()

        e0 = (cid * NTILE + sid) * n_per_worker

        def step(i, _):
            off = e0 + i * sup
            pltpu.sync_copy(src_ref.at[pl.ds(off, sup)], idx_v)
            pltpu.sync_copy(dst_ref.at[pl.ds(e0 // CHUNK + i * SUP, SUP)],
                            dst_v)
            gs = [pltpu.async_copy(
                      table_ref.at[idx_v.at[pl.ds(j * CHUNK, CHUNK)]],
                      stage_v.at[j], gsem) for j in range(SUP)]
            for g in gs:
                g.wait()
            ss = [pltpu.async_copy(stage_v.at[j], acc_sh.at[dst_v.at[j]],
                                   ssem, add=True) for j in range(SUP)]
            for sc in ss:
                sc.wait()
            return 0

        lax.fori_loop(0, n_chunks, step, 0)
        plsc.subcore_barrier()

        @pl.when(sid < NTILE - 1)
        def _():
            r0 = sid * 3128
            pltpu.sync_copy(acc_sh.at[pl.ds(r0, 3128)],
                            out_ref.at[pl.ds(cid * N_NODE + r0, 3128)])

        @pl.when(sid == NTILE - 1)
        def _():
            r0 = (NTILE - 1) * 3128
            pltpu.sync_copy(acc_sh.at[pl.ds(r0, N_NODE - r0)],
                            out_ref.at[pl.ds(cid * N_NODE + r0, N_NODE - r0)])

    call = pl.kernel(
        body,
        out_type=jax.ShapeDtypeStruct((2 * N_NODE, 16), jnp.float32),
        mesh=plsc.VectorSubcoreMesh(core_axis_name="c", subcore_axis_name="s"),
        compiler_params=pltpu.CompilerParams(needs_layout_passes=False, use_tc_tiling_on_sc=False),
        scratch_types=[
            pltpu.VMEM_SHARED((NACC, 16), jnp.float32),
            pltpu.VMEM((7 * CHUNK,), jnp.int32),
            pltpu.VMEM((7, CHUNK), jnp.int32),
            pltpu.VMEM((7, CHUNK, 16), jnp.float32),
            pltpu.SemaphoreType.DMA,
            pltpu.SemaphoreType.DMA,
        ],
    )
    return call(table, srcp, dstp.reshape(EPAD // CHUNK, CHUNK),
                jnp.zeros((NACC, 16), jnp.float32))


def _gather_rows(table, idx):
    per_w = BATCH // (NSC * NTILE)

    def body(table_ref, idx_ref, out_ref, idx_v, rows_v, sem):
        w = lax.axis_index("c") * NTILE + lax.axis_index("s")
        base = w * per_w
        pltpu.sync_copy(idx_ref.at[pl.ds(base, per_w)], idx_v)
        pltpu.async_copy(table_ref.at[idx_v], rows_v, sem).wait()
        pltpu.sync_copy(rows_v, out_ref.at[pl.ds(base, per_w)])

    call = pl.kernel(
        body,
        out_type=jax.ShapeDtypeStruct((BATCH, 16), jnp.float32),
        mesh=plsc.VectorSubcoreMesh(core_axis_name="c", subcore_axis_name="s"),
        compiler_params=pltpu.CompilerParams(needs_layout_passes=False, use_tc_tiling_on_sc=False),
        scratch_types=[
            pltpu.VMEM((per_w,), jnp.int32),
            pltpu.VMEM((per_w, 16), jnp.float32),
            pltpu.SemaphoreType.DMA,
        ],
    )
    return call(table, idx)


def _proj_stacked(x, w):
    n, d = x.shape
    wstk = jnp.stack([w[:, :32], w[:, 32:]])

    def body(x_ref, w_ref, o_ref):
        o_ref[...] = jnp.dot(x_ref[...], w_ref[0],
                             preferred_element_type=jnp.float32)

    nb = n // RBLK
    return pl.pallas_call(
        body,
        grid=(2, nb),
        in_specs=[pl.BlockSpec((RBLK, d), lambda c, i: (i, 0)),
                  pl.BlockSpec((1, d, 32), lambda c, i: (c, 0, 0))],
        out_specs=pl.BlockSpec((RBLK, 32), lambda c, i: (c * nb + i, 0)),
        out_shape=jax.ShapeDtypeStruct((2 * n, 32), jnp.float32),
    )(x, wstk)


def _hist_reduce(hist):
    t, m = hist.shape

    def body(h_ref, o_ref):
        o_ref[...] = jnp.sum(h_ref[...], axis=0, keepdims=True)

    return pl.pallas_call(
        body,
        grid=(m // 6400,),
        in_specs=[pl.BlockSpec((t, 6400), lambda i: (0, i))],
        out_specs=pl.BlockSpec((1, 6400), lambda i: (0, i)),
        out_shape=jax.ShapeDtypeStruct((1, m), jnp.float32),
    )(hist)


def _combine_relu(sa, sb, cnt, x, wr, b):
    n, d = x.shape

    def body(sa_ref, sb_ref, c_ref, x_ref, w_ref, b_ref, o_ref):
        c = jnp.clip(c_ref[...], 1.0)
        m = jnp.concatenate([sa_ref[...], sb_ref[...]], axis=1) / c
        o_ref[...] = jnp.maximum(
            m + b_ref[...] + jnp.dot(x_ref[...], w_ref[...],
                                     preferred_element_type=jnp.float32), 0.0)

    return pl.pallas_call(
        body,
        grid=(n // RBLK,),
        in_specs=[pl.BlockSpec((RBLK, 32), lambda i: (i, 0)),
                  pl.BlockSpec((RBLK, 32), lambda i: (i, 0)),
                  pl.BlockSpec((RBLK, 1), lambda i: (i, 0)),
                  pl.BlockSpec((RBLK, d), lambda i: (i, 0)),
                  pl.BlockSpec((d, DIM_H), lambda i: (0, 0)),
                  pl.BlockSpec((1, DIM_H), lambda i: (0, 0))],
        out_specs=pl.BlockSpec((RBLK, DIM_H), lambda i: (i, 0)),
        out_shape=jax.ShapeDtypeStruct((n, DIM_H), jnp.float32),
    )(sa, sb, cnt, x, wr, b)


def _proj_fold(x, w1, w2):
    n = x.shape[0]

    def body(x_ref, w1_ref, w2_ref, o_ref):
        w = jnp.dot(w1_ref[...], w2_ref[...], preferred_element_type=jnp.float32)
        o_ref[...] = jnp.dot(x_ref[...], w, preferred_element_type=jnp.float32)

    return pl.pallas_call(
        body,
        grid=(n // RBLK,),
        in_specs=[pl.BlockSpec((RBLK, DIM_H), lambda i: (i, 0)),
                  pl.BlockSpec((DIM_H, DIM_H), lambda i: (0, 0)),
                  pl.BlockSpec((DIM_H, 16), lambda i: (0, 0))],
        out_specs=pl.BlockSpec((RBLK, 16), lambda i: (i, 0)),
        out_shape=jax.ShapeDtypeStruct((n, 16), jnp.float32),
    )(x, w1, w2)


def _final_dense(s3a, s3b, hist, hc, wr2, wcp, b2, bcp):
    n = hc.shape[0]

    def body(sa_ref, sb_ref, c_ref, hc_ref, wr_ref, wc_ref, b2_ref, bc_ref,
             o_ref):
        c = jnp.clip(c_ref[...], 1.0)
        m = (sa_ref[...] + sb_ref[...]) / c
        fold = jnp.dot(wr_ref[...], wc_ref[...],
                       preferred_element_type=jnp.float32)
        bias = jnp.dot(b2_ref[...], wc_ref[...],
                       preferred_element_type=jnp.float32) + bc_ref[...]
        o_ref[...] = m + bias + jnp.dot(hc_ref[...], fold,
                                        preferred_element_type=jnp.float32)

    return pl.pallas_call(
        body,
        grid=(n // RBLK,),
        in_specs=[pl.BlockSpec((RBLK, 16), lambda i: (i, 0)),
                  pl.BlockSpec((RBLK, 16), lambda i: (i, 0)),
                  pl.BlockSpec((RBLK, 1), lambda i: (i, 0)),
                  pl.BlockSpec((RBLK, DIM_H), lambda i: (i, 0)),
                  pl.BlockSpec((DIM_H, DIM_H), lambda i: (0, 0)),
                  pl.BlockSpec((DIM_H, 16), lambda i: (0, 0)),
                  pl.BlockSpec((1, DIM_H), lambda i: (0, 0)),
                  pl.BlockSpec((1, 16), lambda i: (0, 0))],
        out_specs=pl.BlockSpec((RBLK, 16), lambda i: (i, 0)),
        out_shape=jax.ShapeDtypeStruct((n, 16), jnp.float32),
    )(s3a, s3b, hist, hc, wr2, wcp, b2, bcp)


def kernel(x_claim, x_entity, edge_index_cm, edge_index_ec, claim_batch_idx,
           W1_cm_l, b1_cm, W1_cm_r, W1_ec_l, b1_ec, W1_ec_r,
           W2_cm_l, b2_cm, W2_cm_r, W2_ec_l, b2_ec, W2_ec_r,
           Wc, bc):
    pad = EPAD - N_EDGE
    i32 = jnp.int32

    def pad_edges(ei):
        src = jnp.concatenate([ei[0].astype(i32), jnp.zeros((pad,), i32)])
        dst = jnp.concatenate([ei[1].astype(i32), jnp.full((pad,), TRASH, i32)])
        return src, dst

    src_cm, dst_cm = pad_edges(edge_index_cm)
    src_ec, dst_ec = pad_edges(edge_index_ec)

    wcp = jnp.pad(Wc, ((0, 0), (0, 14)))
    bcp = jnp.pad(bc, (0, 14)).reshape(1, 16)

    p1c = _proj_stacked(x_claim, W1_cm_l)
    p1e = _proj_stacked(x_entity, W1_ec_l)
    s1 = _seg_sum_feat_split(p1c, src_cm, dst_cm)
    s2 = _seg_sum_feat_split(p1e, src_ec, dst_ec)
    hist_cm, hist_ec = _degree_counts(dst_cm, dst_ec)
    cnt_cm = _hist_reduce(hist_cm.reshape(NSC * NTILE, -1))
    cnt_ec = _hist_reduce(hist_ec.reshape(NSC * NTILE, -1))
    cnt_cm = cnt_cm.reshape(-1)[:N_NODE].reshape(N_NODE, 1)
    cnt_ec = cnt_ec.reshape(-1)[:N_NODE].reshape(N_NODE, 1)
    h_e = _combine_relu(s1[:N_NODE], s1[N_NODE:], cnt_cm, x_entity, W1_cm_r,
                        b1_cm.reshape(1, -1))
    h_c = _combine_relu(s2[:N_NODE], s2[N_NODE:], cnt_ec, x_claim, W1_ec_r,
                        b1_ec.reshape(1, -1))

    p2 = _proj_fold(h_e, W2_ec_l, wcp)
    s3 = _seg_sum_narrow(p2, src_ec, dst_ec)
    out16 = _final_dense(s3[:N_NODE], s3[N_NODE:], cnt_ec, h_c, W2_ec_r, wcp,
                         b2_ec.reshape(1, -1), bcp)
    res = _gather_rows(out16, claim_batch_idx.astype(i32))
    return res[:, :2]
